# baseline (speedup 1.0000x reference)
import functools

import jax
import jax.numpy as jnp
from jax import lax
from jax.experimental import pallas as pl
from jax.experimental.pallas import tpu as pltpu
from jax.experimental.pallas import tpu_sc as plsc

N_NODES = 10000
N_FEAT = 128
N_HID = 64
ALPHA = 0.2
N_EDGES = 320000

ROWW = 80
NPAD = 10240
NW = 32
EPW = NPAD
E_PAD = EPW * NW
CHUNK = 80
NCHUNK = EPW // CHUNK


def _prolog_body(x_ref, w_ref, a_ref, hd_ref, ss_ref):
    h = jnp.dot(x_ref[...], w_ref[...], preferred_element_type=jnp.float32)
    s = jnp.dot(h, a_ref[...], preferred_element_type=jnp.float32)
    hd_ref[...] = jnp.concatenate([h, s[:, 16:32]], axis=1)
    ss_ref[...] = s[:, 0:16]


def _edge_body(src_hbm, dst_hbm, s1_hbm, hd_hbm, out_hbm,
               src_v, dst_v, s1_v, hd_v0, hd_v1, hd_v2, hd_v3,
               si_v0, si_v1, si_v2, si_v3,
               acc, gsem0, gsem1, gsem2, gsem3, ssem0, ssem1, ssem2, ssem3):
    cid = lax.axis_index("c")
    sid = lax.axis_index("s")
    wid = sid * 2 + cid
    hd_v = (hd_v0, hd_v1, hd_v2, hd_v3)
    si_v = (si_v0, si_v1, si_v2, si_v3)
    gsem = (gsem0, gsem1, gsem2, gsem3)
    ssem = (ssem0, ssem1, ssem2, ssem3)
    NB = 4

    base0 = wid * EPW
    pltpu.sync_copy(src_hbm.at[pl.ds(base0, EPW)], src_v)
    pltpu.sync_copy(dst_hbm.at[pl.ds(base0, EPW)], dst_v)
    pltpu.sync_copy(s1_hbm, s1_v)

    zf = jnp.zeros((16,), jnp.float32)

    def zero_row(r, carry):
        for c4 in range(ROWW // 16):
            hd_v0[r, pl.ds(c4 * 16, 16)] = zf
        return carry

    lax.fori_loop(0, CHUNK, zero_row, 0)
    rows_per_sub = NPAD // 16
    for k in range(rows_per_sub // CHUNK):
        pltpu.sync_copy(hd_v0, acc.at[pl.ds(sid * rows_per_sub + k * CHUNK, CHUNK)])
    plsc.subcore_barrier()

    iota = lax.iota(jnp.int32, 16)
    c_w = jnp.full((16,), N_HID, jnp.int32)

    def gather_rows(ch, b):
        h = CHUNK // 2
        pltpu.async_copy(
            hd_hbm.at[dst_v.at[pl.ds(ch * CHUNK, h)]],
            hd_v[b].at[pl.ds(0, h)], gsem[b])
        pltpu.async_copy(
            hd_hbm.at[dst_v.at[pl.ds(ch * CHUNK + h, h)]],
            hd_v[b].at[pl.ds(h, h)], gsem[b])

    def drain(dst_ref, sem):
        pltpu.make_async_copy(hd_hbm.at[pl.ds(0, CHUNK)], dst_ref, sem).wait()

    for pb in range(NB - 1):
        gather_rows(pb, pb)

    def body(i, carry):
        for b in range(NB):
            ch = i * NB + b
            b3 = (b + NB - 1) % NB

            @pl.when(jnp.logical_and(ch > 0, ch + NB - 1 < NCHUNK))
            def _():
                drain(hd_v[b3], ssem[b3])

            @pl.when(ch + NB - 1 < NCHUNK)
            def _():
                gather_rows(ch + NB - 1, b3)

            drain(hd_v[b], gsem[b])

            for g in range(CHUNK // 16):
                row = iota + g * 16
                srcv = src_v[pl.ds(ch * CHUNK + g * 16, 16)]
                si_v[b][pl.ds(g * 16, 16)] = srcv
                s1 = plsc.load_gather(s1_v, [srcv])
                s2 = plsc.load_gather(hd_v[b], [row, c_w])
                e = s1 + s2
                w = jnp.exp(-jnp.maximum(e, ALPHA * e))
                plsc.store_scatter(hd_v[b], [row, c_w], w)
                for c0 in range(0, N_HID, 8):
                    ccs = [jnp.full((16,), c0 + u, jnp.int32) for u in range(8)]
                    cols = [plsc.load_gather(hd_v[b], [row, cc]) for cc in ccs]
                    outs = [col * w for col in cols]
                    for u in range(8):
                        plsc.store_scatter(hd_v[b], [row, ccs[u]], outs[u])
            pltpu.async_copy(hd_v[b], acc.at[si_v[b]], ssem[b], add=True)
        return carry

    lax.fori_loop(0, NCHUNK // NB, body, 0)
    for b in range(NB):
        drain(hd_v[b], ssem[b])
    plsc.subcore_barrier()
    pltpu.sync_copy(acc.at[pl.ds(sid * rows_per_sub, rows_per_sub)],
                    out_hbm.at[cid, pl.ds(sid * rows_per_sub, rows_per_sub)])


_edge_kernel = functools.partial(
    pl.kernel,
    out_type=jax.ShapeDtypeStruct((2, NPAD, ROWW), jnp.float32),
    mesh=plsc.VectorSubcoreMesh(core_axis_name="c", subcore_axis_name="s"),
    compiler_params=pltpu.CompilerParams(
        needs_layout_passes=False, use_tc_tiling_on_sc=False),
    scratch_types=[
        pltpu.VMEM((EPW,), jnp.int32),
        pltpu.VMEM((EPW,), jnp.int32),
        pltpu.VMEM((NPAD,), jnp.float32),
        pltpu.VMEM((CHUNK, ROWW), jnp.float32),
        pltpu.VMEM((CHUNK, ROWW), jnp.float32),
        pltpu.VMEM((CHUNK, ROWW), jnp.float32),
        pltpu.VMEM((CHUNK, ROWW), jnp.float32),
        pltpu.VMEM((CHUNK,), jnp.int32),
        pltpu.VMEM((CHUNK,), jnp.int32),
        pltpu.VMEM((CHUNK,), jnp.int32),
        pltpu.VMEM((CHUNK,), jnp.int32),
        pltpu.VMEM_SHARED((NPAD, ROWW), jnp.float32),
        pltpu.SemaphoreType.DMA,
        pltpu.SemaphoreType.DMA,
        pltpu.SemaphoreType.DMA,
        pltpu.SemaphoreType.DMA,
        pltpu.SemaphoreType.DMA,
        pltpu.SemaphoreType.DMA,
        pltpu.SemaphoreType.DMA,
        pltpu.SemaphoreType.DMA,
    ],
)(_edge_body)


def _final_body(p_ref, o_ref):
    p = p_ref[...]
    acc = p[0] + p[1]
    num = acc[:, :N_HID]
    den = acc[:, N_HID:N_HID + 1] + 1e-8
    hp = num / den
    o_ref[...] = jnp.where(hp > 0, hp, jnp.exp(hp) - 1.0)


def kernel(x, edge_index, W, a):
    src = edge_index[0]
    dst = edge_index[1]
    n_pad = E_PAD - N_EDGES
    src_p = jnp.concatenate([src, jnp.full((n_pad,), N_NODES, jnp.int32)])
    dst_p = jnp.concatenate([dst, jnp.zeros((n_pad,), jnp.int32)])

    a_pack = jnp.zeros((N_HID, 32), jnp.float32)
    a_pack = a_pack.at[:, 0].set(a[0, :N_HID]).at[:, 16].set(a[0, N_HID:])

    blk = 1000
    hd, ssrc = pl.pallas_call(
        _prolog_body,
        grid=(N_NODES // blk,),
        in_specs=[
            pl.BlockSpec((blk, N_FEAT), lambda i: (i, 0)),
            pl.BlockSpec((N_FEAT, N_HID), lambda i: (0, 0)),
            pl.BlockSpec((N_HID, 32), lambda i: (0, 0)),
        ],
        out_specs=[
            pl.BlockSpec((blk, ROWW), lambda i: (i, 0)),
            pl.BlockSpec((blk, 16), lambda i: (i, 0)),
        ],
        out_shape=[
            jax.ShapeDtypeStruct((N_NODES, ROWW), jnp.float32),
            jax.ShapeDtypeStruct((N_NODES, 16), jnp.float32),
        ],
    )(x, W, a_pack)

    s1_flat = jnp.pad(ssrc[:, 0], (0, NPAD - N_NODES))

    partials = _edge_kernel(src_p, dst_p, s1_flat, hd)

    blk2 = 400
    return pl.pallas_call(
        _final_body,
        grid=(N_NODES // blk2,),
        in_specs=[pl.BlockSpec((2, blk2, ROWW), lambda i: (0, i, 0))],
        out_specs=pl.BlockSpec((blk2, N_HID), lambda i: (i, 0)),
        out_shape=jax.ShapeDtypeStruct((N_NODES, N_HID), jnp.float32),
    )(partials)


# speedup vs baseline: 1.0438x; 1.0438x over previous
import functools

import jax
import jax.numpy as jnp
from jax import lax
from jax.experimental import pallas as pl
from jax.experimental.pallas import tpu as pltpu
from jax.experimental.pallas import tpu_sc as plsc

N_NODES = 10000
N_FEAT = 128
N_HID = 64
ALPHA = 0.2
N_EDGES = 320000

ROWW = 72
NPAD = 10240
NW = 32
EPW = NPAD
E_PAD = EPW * NW
CHUNK = 80
NCHUNK = EPW // CHUNK


def _prolog_body(x_ref, w_ref, a_ref, hd_ref, ss_ref):
    h = jnp.dot(x_ref[...], w_ref[...], preferred_element_type=jnp.float32)
    s = jnp.dot(h, a_ref[...], preferred_element_type=jnp.float32)
    hd_ref[...] = jnp.concatenate([h, s[:, 16:24]], axis=1)
    ss_ref[...] = s[:, 0:16]


def _edge_body(src_hbm, dst_hbm, s1_hbm, hd_hbm, out_hbm,
               src_v, dst_v, s1_v, hd_v0, hd_v1, hd_v2, hd_v3,
               si_v0, si_v1, si_v2, si_v3,
               acc, gsem0, gsem1, gsem2, gsem3, ssem0, ssem1, ssem2, ssem3):
    cid = lax.axis_index("c")
    sid = lax.axis_index("s")
    wid = sid * 2 + cid
    hd_v = (hd_v0, hd_v1, hd_v2, hd_v3)
    si_v = (si_v0, si_v1, si_v2, si_v3)
    gsem = (gsem0, gsem1, gsem2, gsem3)
    ssem = (ssem0, ssem1, ssem2, ssem3)
    NB = 4

    base0 = wid * EPW
    pltpu.sync_copy(src_hbm.at[pl.ds(base0, EPW)], src_v)
    pltpu.sync_copy(dst_hbm.at[pl.ds(base0, EPW)], dst_v)
    pltpu.sync_copy(s1_hbm, s1_v)

    zf = jnp.zeros((16,), jnp.float32)

    def zero_row(r, carry):
        for off in (0, 16, 32, 48, ROWW - 16):
            hd_v0[r, pl.ds(off, 16)] = zf
        return carry

    lax.fori_loop(0, CHUNK, zero_row, 0)
    rows_per_sub = NPAD // 16
    for k in range(rows_per_sub // CHUNK):
        pltpu.sync_copy(hd_v0, acc.at[pl.ds(sid * rows_per_sub + k * CHUNK, CHUNK)])
    plsc.subcore_barrier()

    iota = lax.iota(jnp.int32, 16)
    c_w = jnp.full((16,), N_HID, jnp.int32)

    def gather_rows(ch, b):
        h = CHUNK // 2
        pltpu.async_copy(
            hd_hbm.at[dst_v.at[pl.ds(ch * CHUNK, h)]],
            hd_v[b].at[pl.ds(0, h)], gsem[b])
        pltpu.async_copy(
            hd_hbm.at[dst_v.at[pl.ds(ch * CHUNK + h, h)]],
            hd_v[b].at[pl.ds(h, h)], gsem[b])

    def drain(dst_ref, sem):
        pltpu.make_async_copy(hd_hbm.at[pl.ds(0, CHUNK)], dst_ref, sem).wait()

    for pb in range(NB - 1):
        gather_rows(pb, pb)

    def body(i, carry):
        for b in range(NB):
            ch = i * NB + b
            b3 = (b + NB - 1) % NB

            @pl.when(jnp.logical_and(ch > 0, ch + NB - 1 < NCHUNK))
            def _():
                drain(hd_v[b3], ssem[b3])

            @pl.when(ch + NB - 1 < NCHUNK)
            def _():
                gather_rows(ch + NB - 1, b3)

            drain(hd_v[b], gsem[b])

            for g in range(CHUNK // 16):
                row = iota + g * 16
                srcv = src_v[pl.ds(ch * CHUNK + g * 16, 16)]
                si_v[b][pl.ds(g * 16, 16)] = srcv
                s1 = plsc.load_gather(s1_v, [srcv])
                s2 = plsc.load_gather(hd_v[b], [row, c_w])
                e = s1 + s2
                w = jnp.exp(-jnp.maximum(e, ALPHA * e))
                plsc.store_scatter(hd_v[b], [row, c_w], w)
                for c0 in range(0, N_HID, 8):
                    ccs = [jnp.full((16,), c0 + u, jnp.int32) for u in range(8)]
                    cols = [plsc.load_gather(hd_v[b], [row, cc]) for cc in ccs]
                    outs = [col * w for col in cols]
                    for u in range(8):
                        plsc.store_scatter(hd_v[b], [row, ccs[u]], outs[u])
            pltpu.async_copy(hd_v[b], acc.at[si_v[b]], ssem[b], add=True)
        return carry

    lax.fori_loop(0, NCHUNK // NB, body, 0)
    for b in range(NB):
        drain(hd_v[b], ssem[b])
    plsc.subcore_barrier()
    pltpu.sync_copy(acc.at[pl.ds(sid * rows_per_sub, rows_per_sub)],
                    out_hbm.at[cid, pl.ds(sid * rows_per_sub, rows_per_sub)])


_edge_kernel = functools.partial(
    pl.kernel,
    out_type=jax.ShapeDtypeStruct((2, NPAD, ROWW), jnp.float32),
    mesh=plsc.VectorSubcoreMesh(core_axis_name="c", subcore_axis_name="s"),
    compiler_params=pltpu.CompilerParams(
        needs_layout_passes=False, use_tc_tiling_on_sc=False),
    scratch_types=[
        pltpu.VMEM((EPW,), jnp.int32),
        pltpu.VMEM((EPW,), jnp.int32),
        pltpu.VMEM((NPAD,), jnp.float32),
        pltpu.VMEM((CHUNK, ROWW), jnp.float32),
        pltpu.VMEM((CHUNK, ROWW), jnp.float32),
        pltpu.VMEM((CHUNK, ROWW), jnp.float32),
        pltpu.VMEM((CHUNK, ROWW), jnp.float32),
        pltpu.VMEM((CHUNK,), jnp.int32),
        pltpu.VMEM((CHUNK,), jnp.int32),
        pltpu.VMEM((CHUNK,), jnp.int32),
        pltpu.VMEM((CHUNK,), jnp.int32),
        pltpu.VMEM_SHARED((NPAD, ROWW), jnp.float32),
        pltpu.SemaphoreType.DMA,
        pltpu.SemaphoreType.DMA,
        pltpu.SemaphoreType.DMA,
        pltpu.SemaphoreType.DMA,
        pltpu.SemaphoreType.DMA,
        pltpu.SemaphoreType.DMA,
        pltpu.SemaphoreType.DMA,
        pltpu.SemaphoreType.DMA,
    ],
)(_edge_body)


def _final_body(p_ref, o_ref):
    p = p_ref[...]
    acc = p[0] + p[1]
    num = acc[:, :N_HID]
    den = acc[:, N_HID:N_HID + 1] + 1e-8
    hp = num / den
    o_ref[...] = jnp.where(hp > 0, hp, jnp.exp(hp) - 1.0)


def kernel(x, edge_index, W, a):
    src = edge_index[0]
    dst = edge_index[1]
    n_pad = E_PAD - N_EDGES
    src_p = jnp.concatenate([src, jnp.full((n_pad,), N_NODES, jnp.int32)])
    dst_p = jnp.concatenate([dst, jnp.zeros((n_pad,), jnp.int32)])

    a_pack = jnp.zeros((N_HID, 32), jnp.float32)
    a_pack = a_pack.at[:, 0].set(a[0, :N_HID]).at[:, 16].set(a[0, N_HID:])

    blk = 1000
    hd, ssrc = pl.pallas_call(
        _prolog_body,
        grid=(N_NODES // blk,),
        in_specs=[
            pl.BlockSpec((blk, N_FEAT), lambda i: (i, 0)),
            pl.BlockSpec((N_FEAT, N_HID), lambda i: (0, 0)),
            pl.BlockSpec((N_HID, 32), lambda i: (0, 0)),
        ],
        out_specs=[
            pl.BlockSpec((blk, ROWW), lambda i: (i, 0)),
            pl.BlockSpec((blk, 16), lambda i: (i, 0)),
        ],
        out_shape=[
            jax.ShapeDtypeStruct((N_NODES, ROWW), jnp.float32),
            jax.ShapeDtypeStruct((N_NODES, 16), jnp.float32),
        ],
    )(x, W, a_pack)

    s1_flat = jnp.pad(ssrc[:, 0], (0, NPAD - N_NODES))

    partials = _edge_kernel(src_p, dst_p, s1_flat, hd)

    blk2 = 400
    return pl.pallas_call(
        _final_body,
        grid=(N_NODES // blk2,),
        in_specs=[pl.BlockSpec((2, blk2, ROWW), lambda i: (0, i, 0))],
        out_specs=pl.BlockSpec((blk2, N_HID), lambda i: (i, 0)),
        out_shape=jax.ShapeDtypeStruct((N_NODES, N_HID), jnp.float32),
    )(partials)


# speedup vs baseline: 1.0491x; 1.0051x over previous
import functools

import jax
import jax.numpy as jnp
from jax import lax
from jax.experimental import pallas as pl
from jax.experimental.pallas import tpu as pltpu
from jax.experimental.pallas import tpu_sc as plsc

N_NODES = 10000
N_FEAT = 128
N_HID = 64
ALPHA = 0.2
N_EDGES = 320000

ROWW = 72
NPAD = 10240
NW = 32
EPW = NPAD
E_PAD = EPW * NW
CHUNK = 128
NCHUNK = EPW // CHUNK


def _prolog_body(x_ref, w_ref, a_ref, hd_ref, ss_ref):
    h = jnp.dot(x_ref[...], w_ref[...], preferred_element_type=jnp.float32)
    s = jnp.dot(h, a_ref[...], preferred_element_type=jnp.float32)
    hd_ref[...] = jnp.concatenate([h, s[:, 16:24]], axis=1)
    ss_ref[...] = s[:, 0:16]


def _edge_body(src_hbm, dst_hbm, s1_hbm, hd_hbm, out_hbm,
               src_v, dst_v, s1_v, hd_v0, hd_v1,
               si_v0, si_v1,
               acc, gsem0, gsem1, ssem0, ssem1):
    cid = lax.axis_index("c")
    sid = lax.axis_index("s")
    wid = sid * 2 + cid
    hd_v = (hd_v0, hd_v1)
    si_v = (si_v0, si_v1)
    gsem = (gsem0, gsem1)
    ssem = (ssem0, ssem1)
    NB = 2

    base0 = wid * EPW
    pltpu.sync_copy(src_hbm.at[pl.ds(base0, EPW)], src_v)
    pltpu.sync_copy(dst_hbm.at[pl.ds(base0, EPW)], dst_v)
    pltpu.sync_copy(s1_hbm, s1_v)

    zf = jnp.zeros((16,), jnp.float32)

    def zero_row(r, carry):
        for off in (0, 16, 32, 48, ROWW - 16):
            hd_v0[r, pl.ds(off, 16)] = zf
        return carry

    lax.fori_loop(0, CHUNK, zero_row, 0)
    rows_per_sub = NPAD // 16
    for k in range(rows_per_sub // CHUNK):
        pltpu.sync_copy(hd_v0, acc.at[pl.ds(sid * rows_per_sub + k * CHUNK, CHUNK)])
    plsc.subcore_barrier()

    iota = lax.iota(jnp.int32, 16)
    c_w = jnp.full((16,), N_HID, jnp.int32)

    def gather_rows(ch, b):
        pltpu.async_copy(
            hd_hbm.at[dst_v.at[pl.ds(ch * CHUNK, CHUNK)]],
            hd_v[b], gsem[b])

    def drain(dst_ref, sem):
        pltpu.make_async_copy(hd_hbm.at[pl.ds(0, CHUNK)], dst_ref, sem).wait()

    for pb in range(NB - 1):
        gather_rows(pb, pb)

    def body(i, carry):
        for b in range(NB):
            ch = i * NB + b
            b3 = (b + NB - 1) % NB

            @pl.when(jnp.logical_and(ch > 0, ch + NB - 1 < NCHUNK))
            def _():
                drain(hd_v[b3], ssem[b3])

            @pl.when(ch + NB - 1 < NCHUNK)
            def _():
                gather_rows(ch + NB - 1, b3)

            drain(hd_v[b], gsem[b])

            for g in range(CHUNK // 16):
                row = iota + g * 16
                srcv = src_v[pl.ds(ch * CHUNK + g * 16, 16)]
                si_v[b][pl.ds(g * 16, 16)] = srcv
                s1 = plsc.load_gather(s1_v, [srcv])
                s2 = plsc.load_gather(hd_v[b], [row, c_w])
                e = s1 + s2
                w = jnp.exp(-jnp.maximum(e, ALPHA * e))
                plsc.store_scatter(hd_v[b], [row, c_w], w)
                for c0 in range(0, N_HID, 8):
                    ccs = [jnp.full((16,), c0 + u, jnp.int32) for u in range(8)]
                    cols = [plsc.load_gather(hd_v[b], [row, cc]) for cc in ccs]
                    outs = [col * w for col in cols]
                    for u in range(8):
                        plsc.store_scatter(hd_v[b], [row, ccs[u]], outs[u])
            pltpu.async_copy(hd_v[b], acc.at[si_v[b]], ssem[b], add=True)
        return carry

    lax.fori_loop(0, NCHUNK // NB, body, 0)
    for b in range(NB):
        drain(hd_v[b], ssem[b])
    plsc.subcore_barrier()
    pltpu.sync_copy(acc.at[pl.ds(sid * rows_per_sub, rows_per_sub)],
                    out_hbm.at[cid, pl.ds(sid * rows_per_sub, rows_per_sub)])


_edge_kernel = functools.partial(
    pl.kernel,
    out_type=jax.ShapeDtypeStruct((2, NPAD, ROWW), jnp.float32),
    mesh=plsc.VectorSubcoreMesh(core_axis_name="c", subcore_axis_name="s"),
    compiler_params=pltpu.CompilerParams(
        needs_layout_passes=False, use_tc_tiling_on_sc=False),
    scratch_types=[
        pltpu.VMEM((EPW,), jnp.int32),
        pltpu.VMEM((EPW,), jnp.int32),
        pltpu.VMEM((NPAD,), jnp.float32),
        pltpu.VMEM((CHUNK, ROWW), jnp.float32),
        pltpu.VMEM((CHUNK, ROWW), jnp.float32),
        pltpu.VMEM((CHUNK,), jnp.int32),
        pltpu.VMEM((CHUNK,), jnp.int32),
        pltpu.VMEM_SHARED((NPAD, ROWW), jnp.float32),
        pltpu.SemaphoreType.DMA,
        pltpu.SemaphoreType.DMA,
        pltpu.SemaphoreType.DMA,
        pltpu.SemaphoreType.DMA,
    ],
)(_edge_body)


def _final_body(p_ref, o_ref):
    p = p_ref[...]
    acc = p[0] + p[1]
    num = acc[:, :N_HID]
    den = acc[:, N_HID:N_HID + 1] + 1e-8
    hp = num / den
    o_ref[...] = jnp.where(hp > 0, hp, jnp.exp(hp) - 1.0)


def kernel(x, edge_index, W, a):
    src = edge_index[0]
    dst = edge_index[1]
    n_pad = E_PAD - N_EDGES
    src_p = jnp.concatenate([src, jnp.full((n_pad,), N_NODES, jnp.int32)])
    dst_p = jnp.concatenate([dst, jnp.zeros((n_pad,), jnp.int32)])

    a_pack = jnp.zeros((N_HID, 32), jnp.float32)
    a_pack = a_pack.at[:, 0].set(a[0, :N_HID]).at[:, 16].set(a[0, N_HID:])

    blk = 1000
    hd, ssrc = pl.pallas_call(
        _prolog_body,
        grid=(N_NODES // blk,),
        in_specs=[
            pl.BlockSpec((blk, N_FEAT), lambda i: (i, 0)),
            pl.BlockSpec((N_FEAT, N_HID), lambda i: (0, 0)),
            pl.BlockSpec((N_HID, 32), lambda i: (0, 0)),
        ],
        out_specs=[
            pl.BlockSpec((blk, ROWW), lambda i: (i, 0)),
            pl.BlockSpec((blk, 16), lambda i: (i, 0)),
        ],
        out_shape=[
            jax.ShapeDtypeStruct((N_NODES, ROWW), jnp.float32),
            jax.ShapeDtypeStruct((N_NODES, 16), jnp.float32),
        ],
    )(x, W, a_pack)

    s1_flat = jnp.pad(ssrc[:, 0], (0, NPAD - N_NODES))

    partials = _edge_kernel(src_p, dst_p, s1_flat, hd)

    blk2 = 400
    return pl.pallas_call(
        _final_body,
        grid=(N_NODES // blk2,),
        in_specs=[pl.BlockSpec((2, blk2, ROWW), lambda i: (0, i, 0))],
        out_specs=pl.BlockSpec((blk2, N_HID), lambda i: (i, 0)),
        out_shape=jax.ShapeDtypeStruct((N_NODES, N_HID), jnp.float32),
    )(partials)
